# baseline (speedup 1.0000x reference)
import functools

import numpy as np
import jax
import jax.numpy as jnp
from jax import lax
from jax.experimental import pallas as pl
from jax.experimental.pallas import tpu as pltpu
from jax.experimental.pallas import tpu_sc as plsc

SEQ_LEN = 200
OUT_DIM = 64
SCALE = 8.0
GPAD = 208
CHUNK_A = 112
CHUNK_B = 96
LANES = 16


def _pos_encoding(length, output_dim):
    depth = output_dim / 2
    positions = np.arange(length)[:, np.newaxis]
    depths = np.arange(depth)[np.newaxis, :] / depth
    angle_rates = 1 / 10000 ** depths
    angle_rads = positions * angle_rates
    return np.concatenate(
        [np.sin(angle_rads), np.cos(angle_rads)], axis=-1
    ).astype(np.float32)


_PE_CONST = jnp.asarray(_pos_encoding(SEQ_LEN, OUT_DIM).reshape(-1))


def kernel(x, table):
    B, S = x.shape
    V, D = table.shape
    info = plsc.get_sparse_core_info()
    NC, NS = info.num_cores, info.num_subcores
    NW = NC * NS
    RPW = B // NW

    table2 = table.reshape(V // 2, 2 * D)
    x_flat = x.reshape(-1)
    pidx_flat = jnp.pad(x_flat >> 1, (0, 2 * GPAD))
    par_bc = (
        (x_flat & 1).astype(jnp.float32)[:, None]
        * jnp.ones((1, LANES), jnp.float32)
    ).reshape(-1)

    @functools.partial(
        pl.kernel,
        mesh=plsc.VectorSubcoreMesh(core_axis_name="c", subcore_axis_name="s"),
        out_type=jax.ShapeDtypeStruct((B, S, D), jnp.float32),
        scratch_types=[
            pltpu.VMEM((S * D,), jnp.float32),
            pltpu.VMEM((CHUNK_A,), jnp.int32),
            pltpu.VMEM((CHUNK_A,), jnp.int32),
            pltpu.VMEM((CHUNK_B,), jnp.int32),
            pltpu.VMEM((CHUNK_B,), jnp.int32),
            pltpu.VMEM((S * LANES,), jnp.float32),
            pltpu.VMEM((S * LANES,), jnp.float32),
            pltpu.VMEM((GPAD, 2 * D), jnp.float32),
            pltpu.VMEM((GPAD, 2 * D), jnp.float32),
            pltpu.VMEM((S, D), jnp.float32),
            pltpu.VMEM((S, D), jnp.float32),
            pltpu.SemaphoreType.DMA,
            pltpu.SemaphoreType.DMA,
            pltpu.SemaphoreType.DMA,
            pltpu.SemaphoreType.DMA,
        ],
    )
    def run(
        table_hbm,
        pidx_hbm,
        par_hbm,
        pe_hbm,
        out_hbm,
        pe_v,
        pa0,
        pa1,
        pb0,
        pb1,
        parv0,
        parv1,
        rows0,
        rows1,
        obuf0,
        obuf1,
        gsem0,
        gsem1,
        wsem0,
        wsem1,
    ):
        pa = (pa0, pa1)
        pb = (pb0, pb1)
        parv = (parv0, parv1)
        rows = (rows0, rows1)
        obuf = (obuf0, obuf1)
        gsem = (gsem0, gsem1)
        wsem = (wsem0, wsem1)

        wid = lax.axis_index("s") * NC + lax.axis_index("c")
        base = wid * RPW
        pltpu.sync_copy(pe_hbm, pe_v)

        def gather_descs(b):
            return (
                pltpu.make_async_copy(
                    table_hbm.at[pa[b]], rows[b].at[pl.ds(0, CHUNK_A)], gsem[b]
                ),
                pltpu.make_async_copy(
                    table_hbm.at[pb[b]],
                    rows[b].at[pl.ds(CHUNK_A, CHUNK_B)],
                    gsem[b],
                ),
            )

        def prep(g, b):
            o = (base + g) * S
            pltpu.sync_copy(pidx_hbm.at[pl.ds(o, CHUNK_A)], pa[b])
            pltpu.sync_copy(pidx_hbm.at[pl.ds(o + CHUNK_A, CHUNK_B)], pb[b])
            pltpu.sync_copy(
                par_hbm.at[pl.ds(o * LANES, S * LANES)], parv[b]
            )
            for d in gather_descs(b):
                d.start()

        def write_desc(g, b):
            return pltpu.make_async_copy(
                obuf[b], out_hbm.at[base + g], wsem[b]
            )

        prep(0, 0)

        @pl.loop(0, RPW // 2)
        def _(j):
            for b in range(2):
                r = 2 * j + b
                nb = 1 - b

                @pl.when(r >= 2)
                def _():
                    write_desc(r - 2, b).wait()

                @pl.when(r + 1 < RPW)
                def _():
                    prep(r + 1, nb)

                for d in gather_descs(b):
                    d.wait()

                @plsc.parallel_loop(0, S, unroll=2)
                def _(s):
                    pf = parv[b][pl.ds(s * LANES, LANES)]
                    for k in range(D // LANES):
                        lo = rows[b][s, pl.ds(k * LANES, LANES)]
                        hi = rows[b][s, pl.ds(D + k * LANES, LANES)]
                        v = lo + pf * (hi - lo)
                        obuf[b][s, pl.ds(k * LANES, LANES)] = v * SCALE + (
                            pe_v[pl.ds(s * D + k * LANES, LANES)]
                        )

                write_desc(r, b).start()

        write_desc(RPW - 2, 0).wait()
        write_desc(RPW - 1, 1).wait()

    return run(table2, pidx_flat, par_bc, _PE_CONST)


# speedup vs baseline: 1.3793x; 1.3793x over previous
import functools

import numpy as np
import jax
import jax.numpy as jnp
from jax import lax
from jax.experimental import pallas as pl
from jax.experimental.pallas import tpu as pltpu
from jax.experimental.pallas import tpu_sc as plsc

SEQ_LEN = 200
OUT_DIM = 64
SCALE = 8.0
CHUNK_A = 104
CHUNK_B = 96
NBUF = 2


def _pos_encoding(length, output_dim):
    depth = output_dim / 2
    positions = np.arange(length)[:, np.newaxis]
    depths = np.arange(depth)[np.newaxis, :] / depth
    angle_rates = 1 / 10000 ** depths
    angle_rads = positions * angle_rates
    return np.concatenate(
        [np.sin(angle_rads), np.cos(angle_rads)], axis=-1
    ).astype(np.float32)


_PE_CONST = jnp.asarray(_pos_encoding(SEQ_LEN, OUT_DIM))


def kernel(x, table):
    B, S = x.shape
    V, D = table.shape
    info = plsc.get_sparse_core_info()
    NC, NS = info.num_cores, info.num_subcores
    NW = NC * NS
    RPW = B // NW

    @functools.partial(
        pl.kernel,
        mesh=plsc.VectorSubcoreMesh(core_axis_name="c", subcore_axis_name="s"),
        compiler_params=pltpu.CompilerParams(use_tc_tiling_on_sc=False),
        out_type=jax.ShapeDtypeStruct((B * S * D // 128, 128), jnp.float32),
        scratch_types=[
            pltpu.VMEM((RPW, S), jnp.int32),
            pltpu.VMEM((S, D), jnp.float32),
        ]
        + [pltpu.VMEM((S, D), jnp.float32) for _ in range(NBUF)]
        + [pltpu.VMEM((S * D // 128, 128), jnp.float32) for _ in range(NBUF)]
        + [pltpu.SemaphoreType.DMA for _ in range(2 * NBUF)],
    )
    def run(table_hbm, x_hbm, pe_hbm, out_hbm, idx_all, pe_v, *bufs_and_sems):
        rows = bufs_and_sems[:NBUF]
        obuf = bufs_and_sems[NBUF : 2 * NBUF]
        gsem = bufs_and_sems[2 * NBUF : 3 * NBUF]
        wsem = bufs_and_sems[3 * NBUF : 4 * NBUF]

        wid = lax.axis_index("s") * NC + lax.axis_index("c")
        base = wid * RPW
        pltpu.sync_copy(x_hbm.at[pl.ds(base, RPW)], idx_all)
        pltpu.sync_copy(pe_hbm, pe_v)

        def gather_descs(g, b):
            return (
                pltpu.make_async_copy(
                    table_hbm.at[idx_all.at[g, pl.ds(0, CHUNK_A)]],
                    rows[b].at[pl.ds(0, CHUNK_A)],
                    gsem[b],
                ),
                pltpu.make_async_copy(
                    table_hbm.at[idx_all.at[g, pl.ds(CHUNK_A, CHUNK_B)]],
                    rows[b].at[pl.ds(CHUNK_A, CHUNK_B)],
                    gsem[b],
                ),
            )

        HS = S * D // 128

        def write_desc(g, b):
            return pltpu.make_async_copy(
                obuf[b],
                out_hbm.at[pl.ds((base + g) * HS, HS)],
                wsem[b],
            )

        for d in gather_descs(0, 0):
            d.start()

        @pl.loop(0, RPW // NBUF)
        def _(j):
            for b in range(NBUF):
                g = j * NBUF + b
                nb = (b + 1) % NBUF

                @pl.when(g >= NBUF - 1)
                def _():
                    write_desc(g - (NBUF - 1), nb).wait()

                @pl.when(g + 1 < RPW)
                def _():
                    for d in gather_descs(g + 1, nb):
                        d.start()

                for d in gather_descs(g, b):
                    d.wait()

                @plsc.parallel_loop(0, S, unroll=8)
                def _(s):
                    orow = s >> 1
                    ocol = (s & 1) * D
                    for k in range(D // 16):
                        sl = pl.ds(k * 16, 16)
                        obuf[b][orow, pl.ds(ocol + k * 16, 16)] = (
                            rows[b][s, sl] * SCALE + pe_v[s, sl]
                        )

                write_desc(g, b).start()

        for g in range(RPW - NBUF + 1, RPW):
            write_desc(g, g % NBUF).wait()

    out = run(table, x, _PE_CONST)
    return out.reshape(B, S, D)
